# baseline (speedup 1.0000x reference)
import functools

import jax
import jax.numpy as jnp
from jax import lax
from jax.experimental import pallas as pl
from jax.experimental.pallas import tpu as pltpu
from jax.experimental.pallas import tpu_sc as plsc

VOCAB = 1_000_000
EMB = 32
N_CLS = 5
PROJ = 16
BATCH = 16384
SEQ = 200
HALF = SEQ // 2
NC, NS = 2, 16
NW = NC * NS
ROWS_PER_W = BATCH // NW
CHUNK = ROWS_PER_W // 2
RING = 4

NSLAB = 8
SLAB = 1 << 17
VPAD = NSLAB * SLAB
SBLK = 4096
NBLK = SLAB // SBLK
LAST_BLK = VOCAB // SBLK


def _pack_body(*refs):
    a_refs, w_ref, out_ref = refs[:NSLAB], refs[NSLAB], refs[NSLAB + 1]
    a8 = jnp.concatenate([r[...] for r in a_refs], axis=0)
    out_ref[...] = lax.dot_general(a8, w_ref[...],
                                   (((0,), (0,)), ((), ())),
                                   preferred_element_type=jnp.float32)


def _pack_table(emb_t, w8):
    in_specs = [
        pl.BlockSpec((EMB, SBLK),
                     lambda j, s=s: (0, jnp.minimum(s * NBLK + j, LAST_BLK)))
        for s in range(NSLAB)
    ] + [pl.BlockSpec((NSLAB * EMB, 128), lambda j: (0, 0))]
    return pl.pallas_call(
        _pack_body,
        grid=(NBLK,),
        in_specs=in_specs,
        out_specs=pl.BlockSpec((SBLK, 128), lambda j: (j, 0)),
        out_shape=jax.ShapeDtypeStruct((SLAB, 128), jnp.float32),
    )(*([emb_t] * NSLAB), w8)


_mesh = plsc.VectorSubcoreMesh(core_axis_name="c", subcore_axis_name="s")


NQ = 4
QROWS = ROWS_PER_W // NQ
QX = QROWS * SEQ // 128


@functools.partial(
    pl.kernel,
    out_type=jax.ShapeDtypeStruct((BATCH, PROJ), jnp.float32),
    mesh=_mesh,
    scratch_types=[
        pltpu.VMEM((QX, 128), jnp.int32),
        pltpu.VMEM((QROWS * SEQ,), jnp.int32),
        pltpu.VMEM((RING, SEQ, PROJ), jnp.float32),
        pltpu.VMEM((ROWS_PER_W, PROJ), jnp.float32),
        pltpu.VMEM((PROJ,), jnp.float32),
    ] + [pltpu.SemaphoreType.DMA] * RING,
    compiler_params=pltpu.CompilerParams(use_tc_tiling_on_sc=False),
)
def _sc_pool(tab_hbm, x_hbm, bias_hbm, out_hbm,
             idx_v, idxt_v, rows_v, out_v, bias_v, *sems):
    wid = lax.axis_index("s") * NC + lax.axis_index("c")
    xbase = wid * (ROWS_PER_W * SEQ // 128)

    pltpu.sync_copy(bias_hbm, bias_v)
    bias = bias_v[...]

    def issue(row_in_q, slot):
        pltpu.async_copy(tab_hbm.at[idxt_v.at[pl.ds(row_in_q * SEQ, SEQ)]],
                         rows_v.at[slot], sems[slot])

    def drain(row_in_q, slot):
        pltpu.make_async_copy(tab_hbm.at[idxt_v.at[pl.ds(row_in_q * SEQ, SEQ)]],
                              rows_v.at[slot], sems[slot]).wait()

    for quarter in range(NQ):
        pltpu.sync_copy(x_hbm.at[pl.ds(xbase + quarter * QX, QX)], idx_v)

        def remap(t, carry):
            for u in range(8):
                v = idx_v[t, pl.ds(16 * u, 16)]
                idxt_v[pl.ds(t * 128 + 16 * u, 16)] = (
                    (v & jnp.int32(SLAB - 1)) << 3) | (v >> 17)
            return carry

        lax.fori_loop(0, QX, remap, None)
        for q in range(RING):
            issue(q, q)

        def body(r_outer, carry, quarter=quarter):
            for q in range(RING):
                row = r_outer * RING + q
                drain(row, q)
                accs = [rows_v[q, j] for j in range(4)]
                for j in range(4, SEQ):
                    accs[j % 4] = accs[j % 4] + rows_v[q, j]
                out_v[quarter * QROWS + row] = ((accs[0] + accs[1])
                                                + (accs[2] + accs[3])) + bias

                @pl.when(row + RING < QROWS)
                def _issue_next(row=row, q=q):
                    issue(row + RING, q)
            return carry

        lax.fori_loop(0, QROWS // RING, body, None)

    pltpu.sync_copy(out_v, out_hbm.at[pl.ds(wid * ROWS_PER_W, ROWS_PER_W)])


def kernel(x, emb_table, fc_w, fc_b):
    emb_t = emb_table.T
    w8 = jnp.zeros((NSLAB * EMB, 128), jnp.float32)
    wt = (fc_w.T / SEQ).astype(jnp.float32)
    for s in range(NSLAB):
        w8 = w8.at[s * EMB:(s + 1) * EMB, s * PROJ:s * PROJ + N_CLS].set(wt)
    packed = _pack_table(emb_t, w8)
    tab = packed.reshape(VPAD, PROJ)
    bias_pad = jnp.zeros((PROJ,), jnp.float32).at[:N_CLS].set(fc_b)
    x2 = x.reshape(BATCH * SEQ // 128, 128)
    out16 = _sc_pool(tab, x2, bias_pad)
    return out16[:, :N_CLS]


# speedup vs baseline: 1.0318x; 1.0318x over previous
import functools

import jax
import jax.numpy as jnp
from jax import lax
from jax.experimental import pallas as pl
from jax.experimental.pallas import tpu as pltpu
from jax.experimental.pallas import tpu_sc as plsc

VOCAB = 1_000_000
EMB = 32
N_CLS = 5
PROJ = 16
BATCH = 16384
SEQ = 200
HALF = SEQ // 2
NC, NS = 2, 16
NW = NC * NS
ROWS_PER_W = BATCH // NW
CHUNK = ROWS_PER_W // 2
RING = 4

NSLAB = 8
SLAB = 1 << 17
VPAD = NSLAB * SLAB
SBLK = 8192
NBLK = SLAB // SBLK
LAST_BLK = VOCAB // SBLK


def _pack_body(*refs):
    a_refs, w_ref, out_ref = refs[:NSLAB], refs[NSLAB], refs[NSLAB + 1]
    a8 = jnp.concatenate([r[...] for r in a_refs], axis=0)
    out_ref[...] = lax.dot_general(a8, w_ref[...],
                                   (((0,), (0,)), ((), ())),
                                   preferred_element_type=jnp.float32)


def _pack_table(emb_t, w8):
    in_specs = [
        pl.BlockSpec((EMB, SBLK),
                     lambda j, s=s: (0, jnp.minimum(s * NBLK + j, LAST_BLK)))
        for s in range(NSLAB)
    ] + [pl.BlockSpec((NSLAB * EMB, 128), lambda j: (0, 0))]
    return pl.pallas_call(
        _pack_body,
        grid=(NBLK,),
        in_specs=in_specs,
        out_specs=pl.BlockSpec((SBLK, 128), lambda j: (j, 0)),
        out_shape=jax.ShapeDtypeStruct((SLAB, 128), jnp.float32),
    )(*([emb_t] * NSLAB), w8)


_mesh = plsc.VectorSubcoreMesh(core_axis_name="c", subcore_axis_name="s")


NQ = 4
QROWS = ROWS_PER_W // NQ
QX = QROWS * SEQ // 128


@functools.partial(
    pl.kernel,
    out_type=jax.ShapeDtypeStruct((BATCH, PROJ), jnp.float32),
    mesh=_mesh,
    scratch_types=[
        pltpu.VMEM((2, QX, 128), jnp.int32),
        pltpu.VMEM((QROWS * SEQ,), jnp.int32),
        pltpu.VMEM((RING, SEQ, PROJ), jnp.float32),
        pltpu.VMEM((ROWS_PER_W, PROJ), jnp.float32),
        pltpu.VMEM((PROJ,), jnp.float32),
    ] + [pltpu.SemaphoreType.DMA] * (RING + 1),
    compiler_params=pltpu.CompilerParams(use_tc_tiling_on_sc=False),
)
def _sc_pool(tab_hbm, x_hbm, bias_hbm, out_hbm,
             idx_v, idxt_v, rows_v, out_v, bias_v, *sems):
    wid = lax.axis_index("s") * NC + lax.axis_index("c")
    xbase = wid * (ROWS_PER_W * SEQ // 128)

    pltpu.sync_copy(bias_hbm, bias_v)
    bias = bias_v[...]

    def issue(row_in_q, slot):
        pltpu.async_copy(tab_hbm.at[idxt_v.at[pl.ds(row_in_q * SEQ, SEQ)]],
                         rows_v.at[slot], sems[slot])

    def drain(row_in_q, slot):
        pltpu.make_async_copy(tab_hbm.at[idxt_v.at[pl.ds(row_in_q * SEQ, SEQ)]],
                              rows_v.at[slot], sems[slot]).wait()

    stage_sem = sems[RING]

    def xsrc(quarter):
        return x_hbm.at[pl.ds(xbase + quarter * QX, QX)]

    for quarter in range(NQ):
        buf = quarter % 2
        if quarter == 0:
            pltpu.sync_copy(xsrc(0), idx_v.at[0])
        else:
            pltpu.make_async_copy(xsrc(quarter), idx_v.at[buf],
                                  stage_sem).wait()
        if quarter + 1 < NQ:
            pltpu.async_copy(xsrc(quarter + 1), idx_v.at[1 - buf], stage_sem)

        def remap(t, carry, buf=buf):
            for u in range(8):
                v = idx_v[buf, t, pl.ds(16 * u, 16)]
                idxt_v[pl.ds(t * 128 + 16 * u, 16)] = (
                    (v & jnp.int32(SLAB - 1)) << 3) | (v >> 17)
            return carry

        lax.fori_loop(0, QX, remap, None)
        for q in range(RING):
            issue(q, q)

        def body(r_outer, carry, quarter=quarter):
            for q in range(RING):
                row = r_outer * RING + q
                drain(row, q)
                accs = [rows_v[q, j] for j in range(4)]
                for j in range(4, SEQ):
                    accs[j % 4] = accs[j % 4] + rows_v[q, j]
                out_v[quarter * QROWS + row] = ((accs[0] + accs[1])
                                                + (accs[2] + accs[3])) + bias

                @pl.when(row + RING < QROWS)
                def _issue_next(row=row, q=q):
                    issue(row + RING, q)
            return carry

        lax.fori_loop(0, QROWS // RING, body, None)

    pltpu.sync_copy(out_v, out_hbm.at[pl.ds(wid * ROWS_PER_W, ROWS_PER_W)])


def kernel(x, emb_table, fc_w, fc_b):
    emb_t = emb_table.T
    w8 = jnp.zeros((NSLAB * EMB, 128), jnp.float32)
    wt = (fc_w.T / SEQ).astype(jnp.float32)
    for s in range(NSLAB):
        w8 = w8.at[s * EMB:(s + 1) * EMB, s * PROJ:s * PROJ + N_CLS].set(wt)
    packed = _pack_table(emb_t, w8)
    tab = packed.reshape(VPAD, PROJ)
    bias_pad = jnp.zeros((PROJ,), jnp.float32).at[:N_CLS].set(fc_b)
    x2 = x.reshape(BATCH * SEQ // 128, 128)
    out16 = _sc_pool(tab, x2, bias_pad)
    return out16[:, :N_CLS]


# speedup vs baseline: 1.0405x; 1.0084x over previous
import functools

import jax
import jax.numpy as jnp
from jax import lax
from jax.experimental import pallas as pl
from jax.experimental.pallas import tpu as pltpu
from jax.experimental.pallas import tpu_sc as plsc

VOCAB = 1_000_000
EMB = 32
N_CLS = 5
PROJ = 16
BATCH = 16384
SEQ = 200
HALF = SEQ // 2
NC, NS = 2, 16
NW = NC * NS
ROWS_PER_W = BATCH // NW
CHUNK = ROWS_PER_W // 2
RING = 4

NSLAB = 8
SLAB = 1 << 17
VPAD = NSLAB * SLAB
SBLK = 16384
NBLK = SLAB // SBLK
LAST_BLK = VOCAB // SBLK


def _pack_body(*refs):
    a_refs, w_ref, out_ref = refs[:NSLAB], refs[NSLAB], refs[NSLAB + 1]
    a8 = jnp.concatenate([r[...] for r in a_refs], axis=0)
    out_ref[...] = lax.dot_general(a8, w_ref[...],
                                   (((0,), (0,)), ((), ())),
                                   preferred_element_type=jnp.float32)


def _pack_table(emb_t, w8):
    in_specs = [
        pl.BlockSpec((EMB, SBLK),
                     lambda j, s=s: (0, jnp.minimum(s * NBLK + j, LAST_BLK)))
        for s in range(NSLAB)
    ] + [pl.BlockSpec((NSLAB * EMB, 128), lambda j: (0, 0))]
    return pl.pallas_call(
        _pack_body,
        grid=(NBLK,),
        in_specs=in_specs,
        out_specs=pl.BlockSpec((SBLK, 128), lambda j: (j, 0)),
        out_shape=jax.ShapeDtypeStruct((SLAB, 128), jnp.float32),
    )(*([emb_t] * NSLAB), w8)


_mesh = plsc.VectorSubcoreMesh(core_axis_name="c", subcore_axis_name="s")


NQ = 4
QROWS = ROWS_PER_W // NQ
QX = QROWS * SEQ // 128


@functools.partial(
    pl.kernel,
    out_type=jax.ShapeDtypeStruct((BATCH, PROJ), jnp.float32),
    mesh=_mesh,
    scratch_types=[
        pltpu.VMEM((2, QX, 128), jnp.int32),
        pltpu.VMEM((QROWS * SEQ,), jnp.int32),
        pltpu.VMEM((RING, SEQ, PROJ), jnp.float32),
        pltpu.VMEM((ROWS_PER_W, PROJ), jnp.float32),
        pltpu.VMEM((PROJ,), jnp.float32),
    ] + [pltpu.SemaphoreType.DMA] * (RING + 1),
    compiler_params=pltpu.CompilerParams(use_tc_tiling_on_sc=False),
)
def _sc_pool(tab_hbm, x_hbm, bias_hbm, out_hbm,
             idx_v, idxt_v, rows_v, out_v, bias_v, *sems):
    wid = lax.axis_index("s") * NC + lax.axis_index("c")
    xbase = wid * (ROWS_PER_W * SEQ // 128)

    pltpu.sync_copy(bias_hbm, bias_v)
    bias = bias_v[...]

    def issue(row_in_q, slot):
        pltpu.async_copy(tab_hbm.at[idxt_v.at[pl.ds(row_in_q * SEQ, SEQ)]],
                         rows_v.at[slot], sems[slot])

    def drain(row_in_q, slot):
        pltpu.make_async_copy(tab_hbm.at[idxt_v.at[pl.ds(row_in_q * SEQ, SEQ)]],
                              rows_v.at[slot], sems[slot]).wait()

    stage_sem = sems[RING]

    def xsrc(quarter):
        return x_hbm.at[pl.ds(xbase + quarter * QX, QX)]

    for quarter in range(NQ):
        buf = quarter % 2
        if quarter == 0:
            pltpu.sync_copy(xsrc(0), idx_v.at[0])
        else:
            pltpu.make_async_copy(xsrc(quarter), idx_v.at[buf],
                                  stage_sem).wait()
        if quarter + 1 < NQ:
            pltpu.async_copy(xsrc(quarter + 1), idx_v.at[1 - buf], stage_sem)

        def remap(t, carry, buf=buf):
            for u in range(8):
                v = idx_v[buf, t, pl.ds(16 * u, 16)]
                idxt_v[pl.ds(t * 128 + 16 * u, 16)] = (
                    (v & jnp.int32(SLAB - 1)) << 3) | (v >> 17)
            return carry

        lax.fori_loop(0, QX, remap, None)
        for q in range(RING):
            issue(q, q)

        def body(r_outer, carry, quarter=quarter):
            for q in range(RING):
                row = r_outer * RING + q
                drain(row, q)
                accs = [rows_v[q, j] for j in range(4)]
                for j in range(4, SEQ):
                    accs[j % 4] = accs[j % 4] + rows_v[q, j]
                out_v[quarter * QROWS + row] = ((accs[0] + accs[1])
                                                + (accs[2] + accs[3])) + bias

                @pl.when(row + RING < QROWS)
                def _issue_next(row=row, q=q):
                    issue(row + RING, q)
            return carry

        lax.fori_loop(0, QROWS // RING, body, None)

    pltpu.sync_copy(out_v, out_hbm.at[pl.ds(wid * ROWS_PER_W, ROWS_PER_W)])


def kernel(x, emb_table, fc_w, fc_b):
    emb_t = emb_table.T
    w8 = jnp.zeros((NSLAB * EMB, 128), jnp.float32)
    wt = (fc_w.T / SEQ).astype(jnp.float32)
    for s in range(NSLAB):
        w8 = w8.at[s * EMB:(s + 1) * EMB, s * PROJ:s * PROJ + N_CLS].set(wt)
    packed = _pack_table(emb_t, w8)
    tab = packed.reshape(VPAD, PROJ)
    bias_pad = jnp.zeros((PROJ,), jnp.float32).at[:N_CLS].set(fc_b)
    x2 = x.reshape(BATCH * SEQ // 128, 128)
    out16 = _sc_pool(tab, x2, bias_pad)
    return out16[:, :N_CLS]
